# baseline (speedup 1.0000x reference)
import functools

import jax
import jax.numpy as jnp
from jax import lax
from jax.experimental import pallas as pl
from jax.experimental.pallas import tpu as pltpu
from jax.experimental.pallas import tpu_sc as plsc

N = 10000
E = 320000
D_IN = 128
D_H = 256
N_CLASSES = 32

NCORES = 2
NSUB = 16
CHUNK = 128
COLS = 64

N_PAD = 10240
ROWS_PS = N_PAD // NSUB
PAD_NODE = N_PAD - 1

K_CH = 2
CH = 160
NG = CH // K_CH
CH_SRC = CH + K_CH
E_PAD = NSUB * CH * CHUNK

BN = 512
NB = N_PAD // BN

_mesh = plsc.VectorSubcoreMesh(core_axis_name="c", subcore_axis_name="s")


@functools.partial(
    pl.kernel,
    mesh=_mesh,
    compiler_params=pltpu.CompilerParams(use_tc_tiling_on_sc=False),
    out_type=jax.ShapeDtypeStruct((NCORES * N_PAD, 16), jnp.float32),
    scratch_types=[
        pltpu.VMEM((NG, K_CH * CHUNK), jnp.int32),
        pltpu.VMEM((K_CH * CHUNK, 16), jnp.float32),
        pltpu.VMEM_SHARED((N_PAD, 16), jnp.float32),
    ],
)
def _deg_kernel(dst_hbm, ones_hbm, zeros_hbm, out_hbm, dst_v, ones_v, deg_sh):
    cid = lax.axis_index("c")
    sid = lax.axis_index("s")
    pltpu.sync_copy(dst_hbm.at[sid], dst_v)
    pltpu.sync_copy(ones_hbm, ones_v)
    pltpu.sync_copy(zeros_hbm, deg_sh.at[pl.ds(sid * ROWS_PS, ROWS_PS)])
    plsc.subcore_barrier()

    def body(j, carry):
        pltpu.sync_copy(ones_v, deg_sh.at[dst_v.at[j]], add=True)
        return carry

    lax.fori_loop(0, NG, body, 0)
    plsc.subcore_barrier()
    pltpu.sync_copy(
        deg_sh.at[pl.ds(sid * ROWS_PS, ROWS_PS)],
        out_hbm.at[pl.ds(cid * N_PAD + sid * ROWS_PS, ROWS_PS)],
    )


@functools.partial(
    pl.kernel,
    mesh=_mesh,
    compiler_params=pltpu.CompilerParams(use_tc_tiling_on_sc=False),
    out_type=jax.ShapeDtypeStruct((NCORES * N_PAD, COLS), jnp.float32),
    scratch_types=[
        pltpu.VMEM((NG + 1, K_CH * CHUNK), jnp.int32),
        pltpu.VMEM((NG, K_CH * CHUNK), jnp.int32),
        pltpu.VMEM((K_CH * CHUNK, COLS), jnp.float32),
        pltpu.VMEM((K_CH * CHUNK, COLS), jnp.float32),
        pltpu.VMEM_SHARED((N_PAD, COLS), jnp.float32),
        pltpu.SemaphoreType.DMA,
        pltpu.SemaphoreType.DMA,
    ],
)
def _agg_kernel(hs_hbm, srcadj_hbm, dst_hbm, out_hbm, src_v, dst_v,
                rows0, rows1, acc_sh, g0, g1):
    cid = lax.axis_index("c")
    sid = lax.axis_index("s")
    wid = cid * NSUB + sid
    pltpu.sync_copy(srcadj_hbm.at[wid], src_v)
    pltpu.sync_copy(dst_hbm.at[sid], dst_v)
    pltpu.sync_copy(
        hs_hbm.at[pl.ds(cid * N_PAD + sid * ROWS_PS, ROWS_PS)],
        acc_sh.at[pl.ds(sid * ROWS_PS, ROWS_PS)],
    )
    plsc.subcore_barrier()

    pltpu.async_copy(hs_hbm.at[src_v.at[0]], rows0, g0)

    def body(i, carry):
        j0 = 2 * i
        pltpu.make_async_copy(hs_hbm.at[src_v.at[j0]], rows0, g0).wait()
        pltpu.async_copy(hs_hbm.at[src_v.at[j0 + 1]], rows1, g1)
        pltpu.make_async_copy(hs_hbm.at[src_v.at[j0 + 1]], rows1, g1).wait()
        pltpu.async_copy(hs_hbm.at[src_v.at[j0 + 2]], rows0, g0)
        return carry

    lax.fori_loop(0, NG // 2, body, 0)
    pltpu.make_async_copy(hs_hbm.at[src_v.at[NG]], rows0, g0).wait()
    plsc.subcore_barrier()
    pltpu.sync_copy(
        acc_sh.at[pl.ds(sid * ROWS_PS, ROWS_PS)],
        out_hbm.at[pl.ds(cid * N_PAD + sid * ROWS_PS, ROWS_PS)],
    )


def _mm1_body(x_ref, w_ref, deg_ref, oa_ref, ob_ref):
    dinv = lax.rsqrt(deg_ref[:, 0:1] + 1.0)
    h = jnp.dot(x_ref[...], w_ref[...], preferred_element_type=jnp.float32)
    hs = h * dinv
    oa_ref[...] = hs[:, :COLS]
    ob_ref[...] = hs[:, COLS:]


def _mm1(xp, W1, deg):
    return pl.pallas_call(
        _mm1_body,
        grid=(NB, NCORES),
        in_specs=[
            pl.BlockSpec((BN, D_IN), lambda i, j: (i, 0)),
            pl.BlockSpec((D_IN, 128), lambda i, j: (0, j)),
            pl.BlockSpec((BN, 16), lambda i, j: (i, 0)),
        ],
        out_specs=[
            pl.BlockSpec((BN, COLS), lambda i, j: (j * NB + i, 0)),
            pl.BlockSpec((BN, COLS), lambda i, j: (j * NB + i, 0)),
        ],
        out_shape=[
            jax.ShapeDtypeStruct((NCORES * N_PAD, COLS), jnp.float32),
            jax.ShapeDtypeStruct((NCORES * N_PAD, COLS), jnp.float32),
        ],
    )(xp, W1, deg)


def _mm2_body(a0_ref, b0_ref, a1_ref, b1_ref, deg_ref, bias_ref, w_ref,
              oa_ref, ob_ref):
    dinv = lax.rsqrt(deg_ref[:, 0:1] + 1.0)
    bias = bias_ref[...]
    h = jnp.dot(jnp.maximum(a0_ref[...] * dinv + bias[0], 0.0),
                w_ref[0:64, :], preferred_element_type=jnp.float32)
    h += jnp.dot(jnp.maximum(b0_ref[...] * dinv + bias[1], 0.0),
                 w_ref[64:128, :], preferred_element_type=jnp.float32)
    h += jnp.dot(jnp.maximum(a1_ref[...] * dinv + bias[2], 0.0),
                 w_ref[128:192, :], preferred_element_type=jnp.float32)
    h += jnp.dot(jnp.maximum(b1_ref[...] * dinv + bias[3], 0.0),
                 w_ref[192:256, :], preferred_element_type=jnp.float32)
    hs = h * dinv
    oa_ref[...] = hs[:, :COLS]
    ob_ref[...] = hs[:, COLS:]


def _mm2(aggA, aggB, deg, b1r, W2):
    return pl.pallas_call(
        _mm2_body,
        grid=(NB, NCORES),
        in_specs=[
            pl.BlockSpec((BN, COLS), lambda i, j: (i, 0)),
            pl.BlockSpec((BN, COLS), lambda i, j: (i, 0)),
            pl.BlockSpec((BN, COLS), lambda i, j: (NB + i, 0)),
            pl.BlockSpec((BN, COLS), lambda i, j: (NB + i, 0)),
            pl.BlockSpec((BN, 16), lambda i, j: (i, 0)),
            pl.BlockSpec((4, 1, COLS), lambda i, j: (0, 0, 0)),
            pl.BlockSpec((D_H, 128), lambda i, j: (0, j)),
        ],
        out_specs=[
            pl.BlockSpec((BN, COLS), lambda i, j: (j * NB + i, 0)),
            pl.BlockSpec((BN, COLS), lambda i, j: (j * NB + i, 0)),
        ],
        out_shape=[
            jax.ShapeDtypeStruct((NCORES * N_PAD, COLS), jnp.float32),
            jax.ShapeDtypeStruct((NCORES * N_PAD, COLS), jnp.float32),
        ],
    )(aggA, aggB, aggA, aggB, deg, b1r, W2)


def _mmc_body(a0_ref, b0_ref, a1_ref, b1_ref, deg_ref, bias_ref, w_ref,
              bc_ref, o_ref):
    dinv = lax.rsqrt(deg_ref[:, 0:1] + 1.0)
    bias = bias_ref[...]
    h = jnp.dot(jnp.maximum(a0_ref[...] * dinv + bias[0], 0.0),
                w_ref[0:64, :], preferred_element_type=jnp.float32)
    h += jnp.dot(jnp.maximum(b0_ref[...] * dinv + bias[1], 0.0),
                 w_ref[64:128, :], preferred_element_type=jnp.float32)
    h += jnp.dot(jnp.maximum(a1_ref[...] * dinv + bias[2], 0.0),
                 w_ref[128:192, :], preferred_element_type=jnp.float32)
    h += jnp.dot(jnp.maximum(b1_ref[...] * dinv + bias[3], 0.0),
                 w_ref[192:256, :], preferred_element_type=jnp.float32)
    o_ref[...] = h + bc_ref[...]


def _mmc(aggA, aggB, deg, b2r, Wc, bcr):
    return pl.pallas_call(
        _mmc_body,
        grid=(NB,),
        in_specs=[
            pl.BlockSpec((BN, COLS), lambda i: (i, 0)),
            pl.BlockSpec((BN, COLS), lambda i: (i, 0)),
            pl.BlockSpec((BN, COLS), lambda i: (NB + i, 0)),
            pl.BlockSpec((BN, COLS), lambda i: (NB + i, 0)),
            pl.BlockSpec((BN, 16), lambda i: (i, 0)),
            pl.BlockSpec((4, 1, COLS), lambda i: (0, 0, 0)),
            pl.BlockSpec((D_H, N_CLASSES), lambda i: (0, 0)),
            pl.BlockSpec((1, N_CLASSES), lambda i: (0, 0)),
        ],
        out_specs=pl.BlockSpec((BN, N_CLASSES), lambda i: (i, 0)),
        out_shape=jax.ShapeDtypeStruct((N_PAD, N_CLASSES), jnp.float32),
    )(aggA, aggB, aggA, aggB, deg, b2r, Wc, bcr)


def kernel(x, edge_index, W1, b1, W2, b2, Wc, bc):
    xp = jnp.zeros((N_PAD, D_IN), jnp.float32).at[:N].set(x)

    src = jnp.concatenate(
        [edge_index[0], jnp.full((E_PAD - E,), PAD_NODE, jnp.int32)])
    dst = jnp.concatenate(
        [edge_index[1], jnp.full((E_PAD - E,), PAD_NODE, jnp.int32)])
    dst_l = dst.reshape(NSUB, NG, K_CH * CHUNK)
    src_l = jnp.concatenate(
        [src.reshape(NSUB, NG, K_CH * CHUNK),
         jnp.full((NSUB, 1, K_CH * CHUNK), PAD_NODE, jnp.int32)], axis=1)
    srcadj = jnp.stack([src_l, src_l + N_PAD]).reshape(
        NCORES * NSUB, NG + 1, K_CH * CHUNK)

    ones_c = jnp.ones((K_CH * CHUNK, 16), jnp.float32)
    zeros_c = jnp.zeros((ROWS_PS, 16), jnp.float32)

    deg = _deg_kernel(dst_l, ones_c, zeros_c)

    hs1A, hs1B = _mm1(xp, W1, deg)
    agg1A = _agg_kernel(hs1A, srcadj, dst_l)
    agg1B = _agg_kernel(hs1B, srcadj, dst_l)

    b1r = b1.reshape(4, 1, COLS)
    hs2A, hs2B = _mm2(agg1A, agg1B, deg, b1r, W2)
    agg2A = _agg_kernel(hs2A, srcadj, dst_l)
    agg2B = _agg_kernel(hs2B, srcadj, dst_l)

    b2r = b2.reshape(4, 1, COLS)
    logits = _mmc(agg2A, agg2B, deg, b2r, Wc, bc.reshape(1, N_CLASSES))
    return logits[:N]


# speedup vs baseline: 1.6365x; 1.6365x over previous
import functools

import jax
import jax.numpy as jnp
from jax import lax
from jax.experimental import pallas as pl
from jax.experimental.pallas import tpu as pltpu
from jax.experimental.pallas import tpu_sc as plsc

N = 10000
E = 320000
D_IN = 128
D_H = 256
N_CLASSES = 32

NCORES = 2
NSUB = 16
CHUNK = 128
COLS = 64

N_PAD = 10240
ROWS_PS = N_PAD // NSUB
PAD_NODE = N_PAD - 1

K_CH = 1
CH = 160
NG = CH // K_CH
E_PAD = NSUB * CH * CHUNK

BN = 512
NB = N_PAD // BN

_mesh = plsc.VectorSubcoreMesh(core_axis_name="c", subcore_axis_name="s")


@functools.partial(
    pl.kernel,
    mesh=_mesh,
    compiler_params=pltpu.CompilerParams(use_tc_tiling_on_sc=False),
    out_type=jax.ShapeDtypeStruct((NCORES * N_PAD, 16), jnp.float32),
    scratch_types=[
        pltpu.VMEM((NG, K_CH * CHUNK), jnp.int32),
        pltpu.VMEM((K_CH * CHUNK, 16), jnp.float32),
        pltpu.VMEM_SHARED((N_PAD, 16), jnp.float32),
    ],
)
def _deg_kernel(dst_hbm, ones_hbm, zeros_hbm, out_hbm, dst_v, ones_v, deg_sh):
    cid = lax.axis_index("c")
    sid = lax.axis_index("s")
    pltpu.sync_copy(dst_hbm.at[sid], dst_v)
    pltpu.sync_copy(ones_hbm, ones_v)
    pltpu.sync_copy(zeros_hbm, deg_sh.at[pl.ds(sid * ROWS_PS, ROWS_PS)])
    plsc.subcore_barrier()

    def body(j, carry):
        pltpu.sync_copy(ones_v, deg_sh.at[dst_v.at[j]], add=True)
        return carry

    lax.fori_loop(0, NG, body, 0)
    plsc.subcore_barrier()
    pltpu.sync_copy(
        deg_sh.at[pl.ds(sid * ROWS_PS, ROWS_PS)],
        out_hbm.at[pl.ds(cid * N_PAD + sid * ROWS_PS, ROWS_PS)],
    )


@functools.partial(
    pl.kernel,
    mesh=_mesh,
    compiler_params=pltpu.CompilerParams(use_tc_tiling_on_sc=False),
    out_type=jax.ShapeDtypeStruct((NCORES * N_PAD, COLS), jnp.float32),
    scratch_types=[
        pltpu.VMEM((NG, K_CH * CHUNK), jnp.int32),
        pltpu.VMEM((NG, K_CH * CHUNK), jnp.int32),
        pltpu.VMEM((K_CH * CHUNK, COLS), jnp.float32),
        pltpu.VMEM_SHARED((N_PAD, COLS), jnp.float32),
        pltpu.VMEM_SHARED((N_PAD, COLS), jnp.float32),
        pltpu.SemaphoreType.DMA,
    ],
)
def _agg_kernel(hs_hbm, src_hbm, dst_hbm, out_hbm, src_v, dst_v,
                rows_v, table_sh, acc_sh, sem):
    cid = lax.axis_index("c")
    sid = lax.axis_index("s")
    pltpu.sync_copy(src_hbm.at[sid], src_v)
    pltpu.sync_copy(dst_hbm.at[sid], dst_v)
    pltpu.sync_copy(
        hs_hbm.at[pl.ds(cid * N_PAD + sid * ROWS_PS, ROWS_PS)],
        table_sh.at[pl.ds(sid * ROWS_PS, ROWS_PS)],
    )
    pltpu.sync_copy(
        hs_hbm.at[pl.ds(cid * N_PAD + sid * ROWS_PS, ROWS_PS)],
        acc_sh.at[pl.ds(sid * ROWS_PS, ROWS_PS)],
    )
    plsc.subcore_barrier()

    def body(i, carry):
        pltpu.async_copy(table_sh.at[src_v.at[i]], rows_v, sem).wait()
        pltpu.sync_copy(rows_v, acc_sh.at[dst_v.at[i]], add=True)
        return carry

    lax.fori_loop(0, NG, body, 0)
    plsc.subcore_barrier()
    pltpu.sync_copy(
        acc_sh.at[pl.ds(sid * ROWS_PS, ROWS_PS)],
        out_hbm.at[pl.ds(cid * N_PAD + sid * ROWS_PS, ROWS_PS)],
    )


def _mm1_body(x_ref, w_ref, deg_ref, oa_ref, ob_ref):
    dinv = lax.rsqrt(deg_ref[:, 0:1] + 1.0)
    h = jnp.dot(x_ref[...], w_ref[...], preferred_element_type=jnp.float32)
    hs = h * dinv
    oa_ref[...] = hs[:, :COLS]
    ob_ref[...] = hs[:, COLS:]


def _mm1(xp, W1, deg):
    return pl.pallas_call(
        _mm1_body,
        grid=(NB, NCORES),
        in_specs=[
            pl.BlockSpec((BN, D_IN), lambda i, j: (i, 0)),
            pl.BlockSpec((D_IN, 128), lambda i, j: (0, j)),
            pl.BlockSpec((BN, 16), lambda i, j: (i, 0)),
        ],
        out_specs=[
            pl.BlockSpec((BN, COLS), lambda i, j: (j * NB + i, 0)),
            pl.BlockSpec((BN, COLS), lambda i, j: (j * NB + i, 0)),
        ],
        out_shape=[
            jax.ShapeDtypeStruct((NCORES * N_PAD, COLS), jnp.float32),
            jax.ShapeDtypeStruct((NCORES * N_PAD, COLS), jnp.float32),
        ],
    )(xp, W1, deg)


def _mm2_body(a0_ref, b0_ref, a1_ref, b1_ref, deg_ref, bias_ref, w_ref,
              oa_ref, ob_ref):
    dinv = lax.rsqrt(deg_ref[:, 0:1] + 1.0)
    bias = bias_ref[...]
    h = jnp.dot(jnp.maximum(a0_ref[...] * dinv + bias[0], 0.0),
                w_ref[0:64, :], preferred_element_type=jnp.float32)
    h += jnp.dot(jnp.maximum(b0_ref[...] * dinv + bias[1], 0.0),
                 w_ref[64:128, :], preferred_element_type=jnp.float32)
    h += jnp.dot(jnp.maximum(a1_ref[...] * dinv + bias[2], 0.0),
                 w_ref[128:192, :], preferred_element_type=jnp.float32)
    h += jnp.dot(jnp.maximum(b1_ref[...] * dinv + bias[3], 0.0),
                 w_ref[192:256, :], preferred_element_type=jnp.float32)
    hs = h * dinv
    oa_ref[...] = hs[:, :COLS]
    ob_ref[...] = hs[:, COLS:]


def _mm2(aggA, aggB, deg, b1r, W2):
    return pl.pallas_call(
        _mm2_body,
        grid=(NB, NCORES),
        in_specs=[
            pl.BlockSpec((BN, COLS), lambda i, j: (i, 0)),
            pl.BlockSpec((BN, COLS), lambda i, j: (i, 0)),
            pl.BlockSpec((BN, COLS), lambda i, j: (NB + i, 0)),
            pl.BlockSpec((BN, COLS), lambda i, j: (NB + i, 0)),
            pl.BlockSpec((BN, 16), lambda i, j: (i, 0)),
            pl.BlockSpec((4, 1, COLS), lambda i, j: (0, 0, 0)),
            pl.BlockSpec((D_H, 128), lambda i, j: (0, j)),
        ],
        out_specs=[
            pl.BlockSpec((BN, COLS), lambda i, j: (j * NB + i, 0)),
            pl.BlockSpec((BN, COLS), lambda i, j: (j * NB + i, 0)),
        ],
        out_shape=[
            jax.ShapeDtypeStruct((NCORES * N_PAD, COLS), jnp.float32),
            jax.ShapeDtypeStruct((NCORES * N_PAD, COLS), jnp.float32),
        ],
    )(aggA, aggB, aggA, aggB, deg, b1r, W2)


def _mmc_body(a0_ref, b0_ref, a1_ref, b1_ref, deg_ref, bias_ref, w_ref,
              bc_ref, o_ref):
    dinv = lax.rsqrt(deg_ref[:, 0:1] + 1.0)
    bias = bias_ref[...]
    h = jnp.dot(jnp.maximum(a0_ref[...] * dinv + bias[0], 0.0),
                w_ref[0:64, :], preferred_element_type=jnp.float32)
    h += jnp.dot(jnp.maximum(b0_ref[...] * dinv + bias[1], 0.0),
                 w_ref[64:128, :], preferred_element_type=jnp.float32)
    h += jnp.dot(jnp.maximum(a1_ref[...] * dinv + bias[2], 0.0),
                 w_ref[128:192, :], preferred_element_type=jnp.float32)
    h += jnp.dot(jnp.maximum(b1_ref[...] * dinv + bias[3], 0.0),
                 w_ref[192:256, :], preferred_element_type=jnp.float32)
    o_ref[...] = h + bc_ref[...]


def _mmc(aggA, aggB, deg, b2r, Wc, bcr):
    return pl.pallas_call(
        _mmc_body,
        grid=(NB,),
        in_specs=[
            pl.BlockSpec((BN, COLS), lambda i: (i, 0)),
            pl.BlockSpec((BN, COLS), lambda i: (i, 0)),
            pl.BlockSpec((BN, COLS), lambda i: (NB + i, 0)),
            pl.BlockSpec((BN, COLS), lambda i: (NB + i, 0)),
            pl.BlockSpec((BN, 16), lambda i: (i, 0)),
            pl.BlockSpec((4, 1, COLS), lambda i: (0, 0, 0)),
            pl.BlockSpec((D_H, N_CLASSES), lambda i: (0, 0)),
            pl.BlockSpec((1, N_CLASSES), lambda i: (0, 0)),
        ],
        out_specs=pl.BlockSpec((BN, N_CLASSES), lambda i: (i, 0)),
        out_shape=jax.ShapeDtypeStruct((N_PAD, N_CLASSES), jnp.float32),
    )(aggA, aggB, aggA, aggB, deg, b2r, Wc, bcr)


def kernel(x, edge_index, W1, b1, W2, b2, Wc, bc):
    xp = jnp.zeros((N_PAD, D_IN), jnp.float32).at[:N].set(x)

    src = jnp.concatenate(
        [edge_index[0], jnp.full((E_PAD - E,), PAD_NODE, jnp.int32)])
    dst = jnp.concatenate(
        [edge_index[1], jnp.full((E_PAD - E,), PAD_NODE, jnp.int32)])
    dst_l = dst.reshape(NSUB, NG, K_CH * CHUNK)
    src_l = src.reshape(NSUB, NG, K_CH * CHUNK)

    ones_c = jnp.ones((K_CH * CHUNK, 16), jnp.float32)
    zeros_c = jnp.zeros((ROWS_PS, 16), jnp.float32)

    deg = _deg_kernel(dst_l, ones_c, zeros_c)

    hs1A, hs1B = _mm1(xp, W1, deg)
    agg1A = _agg_kernel(hs1A, src_l, dst_l)
    agg1B = _agg_kernel(hs1B, src_l, dst_l)

    b1r = b1.reshape(4, 1, COLS)
    hs2A, hs2B = _mm2(agg1A, agg1B, deg, b1r, W2)
    agg2A = _agg_kernel(hs2A, src_l, dst_l)
    agg2B = _agg_kernel(hs2B, src_l, dst_l)

    b2r = b2.reshape(4, 1, COLS)
    logits = _mmc(agg2A, agg2B, deg, b2r, Wc, bc.reshape(1, N_CLASSES))
    return logits[:N]


# speedup vs baseline: 2.0329x; 1.2422x over previous
import functools

import jax
import jax.numpy as jnp
from jax import lax
from jax.experimental import pallas as pl
from jax.experimental.pallas import tpu as pltpu
from jax.experimental.pallas import tpu_sc as plsc

N = 10000
E = 320000
D_IN = 128
D_H = 256
N_CLASSES = 32

NCORES = 2
NSUB = 16
CHUNK = 128
COLS = 64

N_PAD = 10240
ROWS_PS = N_PAD // NSUB
PAD_NODE = N_PAD - 1

CH = 160
NG = CH
GI = 8
NGI = NG // GI
E_PAD = NSUB * CH * CHUNK
GBYTES = CHUNK * COLS * 4
IBYTES = GI * CHUNK * 4

BN = 512
NB = N_PAD // BN

_mesh = plsc.VectorSubcoreMesh(core_axis_name="c", subcore_axis_name="s")


@functools.partial(
    pl.kernel,
    mesh=_mesh,
    compiler_params=pltpu.CompilerParams(use_tc_tiling_on_sc=False),
    out_type=jax.ShapeDtypeStruct((NCORES * N_PAD, 16), jnp.float32),
    scratch_types=[
        pltpu.VMEM((NG, CHUNK), jnp.int32),
        pltpu.VMEM((CHUNK, 16), jnp.float32),
        pltpu.VMEM_SHARED((N_PAD, 16), jnp.float32),
    ],
)
def _deg_kernel(dst_hbm, ones_hbm, zeros_hbm, out_hbm, dst_v, ones_v, deg_sh):
    cid = lax.axis_index("c")
    sid = lax.axis_index("s")
    pltpu.sync_copy(dst_hbm.at[sid, pl.ds(0, NG)], dst_v)
    pltpu.sync_copy(ones_hbm, ones_v)
    pltpu.sync_copy(zeros_hbm, deg_sh.at[pl.ds(sid * ROWS_PS, ROWS_PS)])
    plsc.subcore_barrier()

    def body(j, carry):
        pltpu.sync_copy(ones_v, deg_sh.at[dst_v.at[j]], add=True)
        return carry

    lax.fori_loop(0, NG, body, 0)
    plsc.subcore_barrier()
    pltpu.sync_copy(
        deg_sh.at[pl.ds(sid * ROWS_PS, ROWS_PS)],
        out_hbm.at[pl.ds(cid * N_PAD + sid * ROWS_PS, ROWS_PS)],
    )


@functools.partial(
    pl.kernel,
    mesh=_mesh,
    compiler_params=pltpu.CompilerParams(use_tc_tiling_on_sc=False),
    out_type=jax.ShapeDtypeStruct((NCORES * N_PAD, COLS), jnp.float32),
    scratch_types=[
        pltpu.VMEM((NG + 2, CHUNK), jnp.int32),
        pltpu.VMEM((2 * GI, CHUNK), jnp.int32),
        pltpu.VMEM((CHUNK, COLS), jnp.float32),
        pltpu.VMEM((CHUNK, COLS), jnp.float32),
        pltpu.VMEM_SHARED((N_PAD, COLS), jnp.float32),
        pltpu.VMEM_SHARED((N_PAD, COLS), jnp.float32),
        pltpu.SemaphoreType.DMA,
        pltpu.SemaphoreType.DMA,
        pltpu.SemaphoreType.DMA,
        pltpu.SemaphoreType.DMA,
        pltpu.SemaphoreType.DMA,
    ],
)
def _agg_kernel(hs_hbm, src_hbm, dst_hbm, out_hbm, src_v, dstb,
                buf0, buf1, table_sh, acc_sh, g0, g1, s0, s1, di):
    cid = lax.axis_index("c")
    sid = lax.axis_index("s")
    pltpu.sync_copy(src_hbm.at[sid], src_v)
    pltpu.sync_copy(dst_hbm.at[sid, pl.ds(0, GI)], dstb.at[pl.ds(0, GI)])
    pltpu.sync_copy(
        hs_hbm.at[pl.ds(cid * N_PAD + sid * ROWS_PS, ROWS_PS)],
        table_sh.at[pl.ds(sid * ROWS_PS, ROWS_PS)],
    )
    pltpu.sync_copy(
        hs_hbm.at[pl.ds(cid * N_PAD + sid * ROWS_PS, ROWS_PS)],
        acc_sh.at[pl.ds(sid * ROWS_PS, ROWS_PS)],
    )
    pltpu.async_copy(table_sh.at[src_v.at[0]], buf0, g0)
    pltpu.async_copy(table_sh.at[src_v.at[1]], buf1, g1)
    plsc.subcore_barrier()

    def body(gi, carry):
        p = (gi % 2) * GI
        pn = ((gi + 1) % 2) * GI
        pltpu.async_copy(
            dst_hbm.at[sid, pl.ds((gi + 1) * GI, GI)],
            dstb.at[pl.ds(pn, GI)], di)
        for k in range(GI // 2):
            j0 = gi * GI + 2 * k
            pltpu.make_async_copy(
                table_sh.at[src_v.at[j0]], buf0, g0).wait()
            pltpu.async_copy(buf0, acc_sh.at[dstb.at[p + 2 * k]], s0, add=True)
            pltpu.make_async_copy(
                table_sh.at[src_v.at[j0 + 1]], buf1, g1).wait()
            pltpu.async_copy(buf1, acc_sh.at[dstb.at[p + 2 * k + 1]], s1,
                             add=True)
            pltpu.make_async_copy(
                buf0, acc_sh.at[dstb.at[p + 2 * k]], s0).wait()
            pltpu.async_copy(table_sh.at[src_v.at[j0 + 2]], buf0, g0)
            pltpu.make_async_copy(
                buf1, acc_sh.at[dstb.at[p + 2 * k + 1]], s1).wait()
            pltpu.async_copy(table_sh.at[src_v.at[j0 + 3]], buf1, g1)
        pltpu.make_async_copy(
            dst_hbm.at[sid, pl.ds((gi + 1) * GI, GI)],
            dstb.at[pl.ds(pn, GI)], di).wait()
        return carry

    lax.fori_loop(0, NGI, body, 0)
    pltpu.make_async_copy(table_sh.at[src_v.at[NG]], buf0, g0).wait()
    pltpu.make_async_copy(table_sh.at[src_v.at[NG + 1]], buf1, g1).wait()
    plsc.subcore_barrier()
    pltpu.sync_copy(
        acc_sh.at[pl.ds(sid * ROWS_PS, ROWS_PS)],
        out_hbm.at[pl.ds(cid * N_PAD + sid * ROWS_PS, ROWS_PS)],
    )


def _mm1_body(x_ref, w_ref, deg_ref, oa_ref, ob_ref):
    dinv = lax.rsqrt(deg_ref[:, 0:1] + 1.0)
    h = jnp.dot(x_ref[...], w_ref[...], preferred_element_type=jnp.float32)
    hs = h * dinv
    oa_ref[...] = hs[:, :COLS]
    ob_ref[...] = hs[:, COLS:]


def _mm1(xp, W1, deg):
    return pl.pallas_call(
        _mm1_body,
        grid=(NB, NCORES),
        in_specs=[
            pl.BlockSpec((BN, D_IN), lambda i, j: (i, 0)),
            pl.BlockSpec((D_IN, 128), lambda i, j: (0, j)),
            pl.BlockSpec((BN, 16), lambda i, j: (i, 0)),
        ],
        out_specs=[
            pl.BlockSpec((BN, COLS), lambda i, j: (j * NB + i, 0)),
            pl.BlockSpec((BN, COLS), lambda i, j: (j * NB + i, 0)),
        ],
        out_shape=[
            jax.ShapeDtypeStruct((NCORES * N_PAD, COLS), jnp.float32),
            jax.ShapeDtypeStruct((NCORES * N_PAD, COLS), jnp.float32),
        ],
    )(xp, W1, deg)


def _mm2_body(a0_ref, b0_ref, a1_ref, b1_ref, deg_ref, bias_ref, w_ref,
              oa_ref, ob_ref):
    dinv = lax.rsqrt(deg_ref[:, 0:1] + 1.0)
    bias = bias_ref[...]
    h = jnp.dot(jnp.maximum(a0_ref[...] * dinv + bias[0], 0.0),
                w_ref[0:64, :], preferred_element_type=jnp.float32)
    h += jnp.dot(jnp.maximum(b0_ref[...] * dinv + bias[1], 0.0),
                 w_ref[64:128, :], preferred_element_type=jnp.float32)
    h += jnp.dot(jnp.maximum(a1_ref[...] * dinv + bias[2], 0.0),
                 w_ref[128:192, :], preferred_element_type=jnp.float32)
    h += jnp.dot(jnp.maximum(b1_ref[...] * dinv + bias[3], 0.0),
                 w_ref[192:256, :], preferred_element_type=jnp.float32)
    hs = h * dinv
    oa_ref[...] = hs[:, :COLS]
    ob_ref[...] = hs[:, COLS:]


def _mm2(aggA, aggB, deg, b1r, W2):
    return pl.pallas_call(
        _mm2_body,
        grid=(NB, NCORES),
        in_specs=[
            pl.BlockSpec((BN, COLS), lambda i, j: (i, 0)),
            pl.BlockSpec((BN, COLS), lambda i, j: (i, 0)),
            pl.BlockSpec((BN, COLS), lambda i, j: (NB + i, 0)),
            pl.BlockSpec((BN, COLS), lambda i, j: (NB + i, 0)),
            pl.BlockSpec((BN, 16), lambda i, j: (i, 0)),
            pl.BlockSpec((4, 1, COLS), lambda i, j: (0, 0, 0)),
            pl.BlockSpec((D_H, 128), lambda i, j: (0, j)),
        ],
        out_specs=[
            pl.BlockSpec((BN, COLS), lambda i, j: (j * NB + i, 0)),
            pl.BlockSpec((BN, COLS), lambda i, j: (j * NB + i, 0)),
        ],
        out_shape=[
            jax.ShapeDtypeStruct((NCORES * N_PAD, COLS), jnp.float32),
            jax.ShapeDtypeStruct((NCORES * N_PAD, COLS), jnp.float32),
        ],
    )(aggA, aggB, aggA, aggB, deg, b1r, W2)


def _mmc_body(a0_ref, b0_ref, a1_ref, b1_ref, deg_ref, bias_ref, w_ref,
              bc_ref, o_ref):
    dinv = lax.rsqrt(deg_ref[:, 0:1] + 1.0)
    bias = bias_ref[...]
    h = jnp.dot(jnp.maximum(a0_ref[...] * dinv + bias[0], 0.0),
                w_ref[0:64, :], preferred_element_type=jnp.float32)
    h += jnp.dot(jnp.maximum(b0_ref[...] * dinv + bias[1], 0.0),
                 w_ref[64:128, :], preferred_element_type=jnp.float32)
    h += jnp.dot(jnp.maximum(a1_ref[...] * dinv + bias[2], 0.0),
                 w_ref[128:192, :], preferred_element_type=jnp.float32)
    h += jnp.dot(jnp.maximum(b1_ref[...] * dinv + bias[3], 0.0),
                 w_ref[192:256, :], preferred_element_type=jnp.float32)
    o_ref[...] = h + bc_ref[...]


def _mmc(aggA, aggB, deg, b2r, Wc, bcr):
    return pl.pallas_call(
        _mmc_body,
        grid=(NB,),
        in_specs=[
            pl.BlockSpec((BN, COLS), lambda i: (i, 0)),
            pl.BlockSpec((BN, COLS), lambda i: (i, 0)),
            pl.BlockSpec((BN, COLS), lambda i: (NB + i, 0)),
            pl.BlockSpec((BN, COLS), lambda i: (NB + i, 0)),
            pl.BlockSpec((BN, 16), lambda i: (i, 0)),
            pl.BlockSpec((4, 1, COLS), lambda i: (0, 0, 0)),
            pl.BlockSpec((D_H, N_CLASSES), lambda i: (0, 0)),
            pl.BlockSpec((1, N_CLASSES), lambda i: (0, 0)),
        ],
        out_specs=pl.BlockSpec((BN, N_CLASSES), lambda i: (i, 0)),
        out_shape=jax.ShapeDtypeStruct((N_PAD, N_CLASSES), jnp.float32),
    )(aggA, aggB, aggA, aggB, deg, b2r, Wc, bcr)


def kernel(x, edge_index, W1, b1, W2, b2, Wc, bc):
    xp = jnp.zeros((N_PAD, D_IN), jnp.float32).at[:N].set(x)

    src = jnp.concatenate(
        [edge_index[0], jnp.full((E_PAD - E,), PAD_NODE, jnp.int32)])
    dst = jnp.concatenate(
        [edge_index[1], jnp.full((E_PAD - E,), PAD_NODE, jnp.int32)])
    dst_l = jnp.concatenate(
        [dst.reshape(NSUB, NG, CHUNK),
         jnp.full((NSUB, GI, CHUNK), PAD_NODE, jnp.int32)], axis=1)
    src_l = jnp.concatenate(
        [src.reshape(NSUB, NG, CHUNK),
         jnp.full((NSUB, 2, CHUNK), PAD_NODE, jnp.int32)], axis=1)

    ones_c = jnp.ones((CHUNK, 16), jnp.float32)
    zeros_c = jnp.zeros((ROWS_PS, 16), jnp.float32)

    deg = _deg_kernel(dst_l, ones_c, zeros_c)

    hs1A, hs1B = _mm1(xp, W1, deg)
    agg1A = _agg_kernel(hs1A, src_l, dst_l)
    agg1B = _agg_kernel(hs1B, src_l, dst_l)

    b1r = b1.reshape(4, 1, COLS)
    hs2A, hs2B = _mm2(agg1A, agg1B, deg, b1r, W2)
    agg2A = _agg_kernel(hs2A, src_l, dst_l)
    agg2B = _agg_kernel(hs2B, src_l, dst_l)

    b2r = b2.reshape(4, 1, COLS)
    logits = _mmc(agg2A, agg2B, deg, b2r, Wc, bc.reshape(1, N_CLASSES))
    return logits[:N]
